# Initial kernel scaffold; baseline (speedup 1.0000x reference)
import jax
import jax.numpy as jnp
from jax.experimental import pallas as pl


def kernel(x, y, x_mask, w_lin, b_lin, w_ih, w_hh, b_ih, b_hh):
    raise NotImplementedError("write your pallas kernel here")


# speedup vs baseline: 1.3080x; 1.3080x over previous
import jax
import jax.numpy as jnp
from jax.experimental import pallas as pl
from jax.experimental.pallas import tpu as pltpu

_TB = 32


def _fused_body(x_ref, y_ref, mask_ref, wlin_ref, blin_ref,
                wih_ref, whh_ref, bi_ref, bh_ref,
                xwy_ref, ynew_ref):
    H = y_ref.shape[-1]
    xb = x_ref[...]
    yb = y_ref[...]

    yW = jnp.dot(yb, wlin_ref[...], preferred_element_type=jnp.float32)
    yW = yW + blin_ref[...]

    s = jnp.sum(xb * yW[:, None, :], axis=-1)
    s = jnp.where(mask_ref[...], -jnp.inf, s)
    xwy_ref[...] = s

    m = jnp.max(s, axis=-1, keepdims=True)
    e = jnp.exp(s - m)
    alpha = e * (1.0 / jnp.sum(e, axis=-1, keepdims=True))

    rnn_in = jnp.sum(alpha[:, :, None] * xb, axis=1)

    gi = jnp.dot(rnn_in, wih_ref[...], preferred_element_type=jnp.float32)
    gi = gi + bi_ref[...]
    gh = jnp.dot(yb, whh_ref[...], preferred_element_type=jnp.float32)
    gh = gh + bh_ref[...]
    r = jax.nn.sigmoid(gi[:, :H] + gh[:, :H])
    z = jax.nn.sigmoid(gi[:, H:2 * H] + gh[:, H:2 * H])
    n = jnp.tanh(gi[:, 2 * H:] + r * gh[:, 2 * H:])
    ynew_ref[...] = n + z * (yb - n)


def kernel(x, y, x_mask, w_lin, b_lin, w_ih, w_hh, b_ih, b_hh):
    B, L, D1 = x.shape
    D2 = y.shape[-1]
    H = D2
    TB = _TB if B % _TB == 0 else 8
    grid = (B // TB,)

    b_i = b_ih.reshape(1, 3 * H)
    b_h = b_hh.reshape(1, 3 * H)

    row = lambda shape: pl.BlockSpec(shape, lambda i, _s=shape: (i,) + (0,) * (len(_s) - 1))
    rep = lambda shape: pl.BlockSpec(shape, lambda i, _s=shape: (0,) * len(_s))

    flops = 2 * B * (D2 * D1 + 2 * L * D1 + 3 * D1 * H + 3 * H * H)
    bytes_accessed = 4 * (B * L * D1 + 2 * B * D2 + B * L) + B * L

    xwy, y_new = pl.pallas_call(
        _fused_body,
        out_shape=(jax.ShapeDtypeStruct((B, L), jnp.float32),
                   jax.ShapeDtypeStruct((B, D2), jnp.float32)),
        grid=grid,
        in_specs=[
            row((TB, L, D1)),
            row((TB, D2)),
            row((TB, L)),
            rep((D2, D1)), rep((1, D1)),
            rep((D1, 3 * H)), rep((D2, 3 * H)),
            rep((1, 3 * H)), rep((1, 3 * H)),
        ],
        out_specs=(row((TB, L)), row((TB, D2))),
        compiler_params=pltpu.CompilerParams(
            dimension_semantics=("parallel",),
        ),
        cost_estimate=pl.CostEstimate(flops=flops,
                                      transcendentals=B * (L + 3 * H),
                                      bytes_accessed=bytes_accessed),
    )(x, y, x_mask, w_lin, b_lin, w_ih, w_hh, b_i, b_h)
    return xwy, y_new


# speedup vs baseline: 1.7642x; 1.3488x over previous
import jax
import jax.numpy as jnp
from jax.experimental import pallas as pl
from jax.experimental.pallas import tpu as pltpu

_TB = 32


def _fused_body(x_ref, y_ref, mask_ref, wlin_ref, blin_ref,
                wih_ref, whh_ref, bi_ref, bh_ref,
                xwy_ref, ynew_ref):
    H = y_ref.shape[-1]
    TB, L, D1 = x_ref.shape
    xb = x_ref[...]
    yb = y_ref[...]

    yW = jnp.dot(yb, wlin_ref[...], preferred_element_type=jnp.float32)
    yW = yW + blin_ref[...]

    yW8 = jnp.broadcast_to(yW[:, None, :], (TB, 8, D1))
    prod = xb.reshape(TB, L // 8, 8, D1) * yW8[:, None]
    s = jnp.sum(prod.reshape(TB, L, D1), axis=-1)
    s = jnp.where(mask_ref[...], -jnp.inf, s)
    xwy_ref[...] = s

    m = jnp.max(s, axis=-1, keepdims=True)
    e = jnp.exp(s - m)
    alpha = e * (1.0 / jnp.sum(e, axis=-1, keepdims=True))

    rows = [jnp.dot(alpha[b:b + 1, :], xb[b],
                    preferred_element_type=jnp.float32) for b in range(TB)]
    rnn_in = jnp.concatenate(rows, axis=0)

    gi = jnp.dot(rnn_in, wih_ref[...], preferred_element_type=jnp.float32)
    gi = gi + bi_ref[...]
    gh = jnp.dot(yb, whh_ref[...], preferred_element_type=jnp.float32)
    gh = gh + bh_ref[...]
    r = jax.nn.sigmoid(gi[:, :H] + gh[:, :H])
    z = jax.nn.sigmoid(gi[:, H:2 * H] + gh[:, H:2 * H])
    n = jnp.tanh(gi[:, 2 * H:] + r * gh[:, 2 * H:])
    ynew_ref[...] = n + z * (yb - n)


def kernel(x, y, x_mask, w_lin, b_lin, w_ih, w_hh, b_ih, b_hh):
    B, L, D1 = x.shape
    D2 = y.shape[-1]
    H = D2
    TB = _TB if B % _TB == 0 else 8
    grid = (B // TB,)

    b_i = b_ih.reshape(1, 3 * H)
    b_h = b_hh.reshape(1, 3 * H)

    row = lambda shape: pl.BlockSpec(shape, lambda i, _s=shape: (i,) + (0,) * (len(_s) - 1))
    rep = lambda shape: pl.BlockSpec(shape, lambda i, _s=shape: (0,) * len(_s))

    flops = 2 * B * (D2 * D1 + 2 * L * D1 + 3 * D1 * H + 3 * H * H)
    bytes_accessed = 4 * (B * L * D1 + 2 * B * D2 + B * L) + B * L

    xwy, y_new = pl.pallas_call(
        _fused_body,
        out_shape=(jax.ShapeDtypeStruct((B, L), jnp.float32),
                   jax.ShapeDtypeStruct((B, D2), jnp.float32)),
        grid=grid,
        in_specs=[
            row((TB, L, D1)),
            row((TB, D2)),
            row((TB, L)),
            rep((D2, D1)), rep((1, D1)),
            rep((D1, 3 * H)), rep((D2, 3 * H)),
            rep((1, 3 * H)), rep((1, 3 * H)),
        ],
        out_specs=(row((TB, L)), row((TB, D2))),
        compiler_params=pltpu.CompilerParams(
            dimension_semantics=("arbitrary",),
        ),
        cost_estimate=pl.CostEstimate(flops=flops,
                                      transcendentals=B * (L + 3 * H),
                                      bytes_accessed=bytes_accessed),
    )(x, y, x_mask, w_lin, b_lin, w_ih, w_hh, b_i, b_h)
    return xwy, y_new


# speedup vs baseline: 1.8689x; 1.0593x over previous
import jax
import jax.numpy as jnp
from jax.experimental import pallas as pl
from jax.experimental.pallas import tpu as pltpu

_TB = 64


def _fused_body(x_ref, y_ref, mask_ref, wlin_ref, blin_ref,
                wih_ref, whh_ref, bi_ref, bh_ref,
                xwy_ref, ynew_ref):
    H = y_ref.shape[-1]
    TB, L, D1 = x_ref.shape
    xb = x_ref[...]
    yb = y_ref[...]

    yW = jnp.dot(yb, wlin_ref[...], preferred_element_type=jnp.float32)
    yW = yW + blin_ref[...]

    yW8 = jnp.broadcast_to(yW[:, None, :], (TB, 8, D1))
    prod = xb.reshape(TB, L // 8, 8, D1) * yW8[:, None]
    s = jnp.sum(prod.reshape(TB, L, D1), axis=-1)
    s = jnp.where(mask_ref[...], -jnp.inf, s)
    xwy_ref[...] = s

    m = jnp.max(s, axis=-1, keepdims=True)
    e = jnp.exp(s - m)
    alpha = e * (1.0 / jnp.sum(e, axis=-1, keepdims=True))

    rows = [jnp.dot(alpha[b:b + 1, :], xb[b],
                    preferred_element_type=jnp.float32) for b in range(TB)]
    rnn_in = jnp.concatenate(rows, axis=0)

    gi = jnp.dot(rnn_in, wih_ref[...], preferred_element_type=jnp.float32)
    gi = gi + bi_ref[...]
    gh = jnp.dot(yb, whh_ref[...], preferred_element_type=jnp.float32)
    gh = gh + bh_ref[...]
    r = jax.nn.sigmoid(gi[:, :H] + gh[:, :H])
    z = jax.nn.sigmoid(gi[:, H:2 * H] + gh[:, H:2 * H])
    n = jnp.tanh(gi[:, 2 * H:] + r * gh[:, 2 * H:])
    ynew_ref[...] = n + z * (yb - n)


def kernel(x, y, x_mask, w_lin, b_lin, w_ih, w_hh, b_ih, b_hh):
    B, L, D1 = x.shape
    D2 = y.shape[-1]
    H = D2
    TB = _TB if B % _TB == 0 else 8
    grid = (B // TB,)

    b_i = b_ih.reshape(1, 3 * H)
    b_h = b_hh.reshape(1, 3 * H)

    row = lambda shape: pl.BlockSpec(shape, lambda i, _s=shape: (i,) + (0,) * (len(_s) - 1))
    rep = lambda shape: pl.BlockSpec(shape, lambda i, _s=shape: (0,) * len(_s))

    flops = 2 * B * (D2 * D1 + 2 * L * D1 + 3 * D1 * H + 3 * H * H)
    bytes_accessed = 4 * (B * L * D1 + 2 * B * D2 + B * L) + B * L

    xwy, y_new = pl.pallas_call(
        _fused_body,
        out_shape=(jax.ShapeDtypeStruct((B, L), jnp.float32),
                   jax.ShapeDtypeStruct((B, D2), jnp.float32)),
        grid=grid,
        in_specs=[
            row((TB, L, D1)),
            row((TB, D2)),
            row((TB, L)),
            rep((D2, D1)), rep((1, D1)),
            rep((D1, 3 * H)), rep((D2, 3 * H)),
            rep((1, 3 * H)), rep((1, 3 * H)),
        ],
        out_specs=(row((TB, L)), row((TB, D2))),
        compiler_params=pltpu.CompilerParams(
            dimension_semantics=("arbitrary",),
        ),
        cost_estimate=pl.CostEstimate(flops=flops,
                                      transcendentals=B * (L + 3 * H),
                                      bytes_accessed=bytes_accessed),
    )(x, y, x_mask, w_lin, b_lin, w_ih, w_hh, b_i, b_h)
    return xwy, y_new
